# Initial kernel scaffold; baseline (speedup 1.0000x reference)
import jax
import jax.numpy as jnp
from jax.experimental import pallas as pl


def kernel(x, weight):
    raise NotImplementedError("write your pallas kernel here")


# speedup vs baseline: 2.3902x; 2.3902x over previous
import functools
import math

import jax
import jax.numpy as jnp
from jax import lax
from jax.experimental import pallas as pl
from jax.experimental.pallas import tpu as pltpu
from jax.experimental.pallas import tpu_sc as plsc

NUM_EMB = 100000
DIM = 128
_SCALE = math.sqrt(DIM)

NC = 2
NS = 16
NW = NC * NS

B = 4096 * 50
B_PER_W = B // NW
CHUNK = 128
N_CHUNKS = B_PER_W // CHUNK


def _make_lookup():
    mesh = plsc.VectorSubcoreMesh(core_axis_name="c", subcore_axis_name="s")

    @functools.partial(
        pl.kernel,
        mesh=mesh,
        out_type=jax.ShapeDtypeStruct((B, DIM), jnp.float32),
        scratch_types=[
            pltpu.VMEM((N_CHUNKS, CHUNK), jnp.int32),
            pltpu.VMEM((CHUNK, DIM), jnp.float32),
            pltpu.SemaphoreType.DMA,
        ],
    )
    def lookup(idx_hbm, table_hbm, out_hbm, idx_v, rows_v, sem):
        wid = lax.axis_index("s") * NC + lax.axis_index("c")
        pltpu.sync_copy(idx_hbm.at[wid], idx_v)

        def chunk_body(j, carry):
            pltpu.async_copy(table_hbm.at[idx_v.at[j]], rows_v, sem).wait()

            def row_body(r, c2):
                for c in range(DIM // 16):
                    sl = (r, pl.ds(c * 16, 16))
                    rows_v[sl] = rows_v[sl] * _SCALE
                return c2

            lax.fori_loop(0, CHUNK, row_body, 0)
            pltpu.sync_copy(
                rows_v, out_hbm.at[pl.ds((wid * N_CHUNKS + j) * CHUNK, CHUNK)]
            )
            return carry

        lax.fori_loop(0, N_CHUNKS, chunk_body, 0)

    return lookup


_lookup = _make_lookup()


@jax.jit
def kernel(x, weight):
    idx = x.reshape(NW, N_CHUNKS, CHUNK)
    out = _lookup(idx, weight)
    return out.reshape(x.shape[0], x.shape[1], DIM)


# speedup vs baseline: 2.9612x; 1.2389x over previous
import functools
import math

import jax
import jax.numpy as jnp
from jax import lax
from jax.experimental import pallas as pl
from jax.experimental.pallas import tpu as pltpu
from jax.experimental.pallas import tpu_sc as plsc

NUM_EMB = 100000
DIM = 128
_SCALE = math.sqrt(DIM)

NC = 2
NS = 16
NW = NC * NS

B = 4096 * 50
B_PER_W = B // NW
CHUNK = 128
N_CHUNKS = B_PER_W // CHUNK
NB = 5
G = 3


def _make_lookup():
    mesh = plsc.VectorSubcoreMesh(core_axis_name="c", subcore_axis_name="s")

    @functools.partial(
        pl.kernel,
        mesh=mesh,
        out_type=jax.ShapeDtypeStruct((B, DIM), jnp.float32),
        scratch_types=[
            pltpu.VMEM((N_CHUNKS, CHUNK), jnp.int32),
            pltpu.VMEM((NB, CHUNK, DIM), jnp.float32),
            pltpu.SemaphoreType.DMA((NB,)),
            pltpu.SemaphoreType.DMA((NB,)),
        ],
    )
    def lookup(idx_hbm, table_hbm, out_hbm, idx_v, rows_v, gsem, osem):
        wid = lax.axis_index("s") * NC + lax.axis_index("c")
        pltpu.sync_copy(idx_hbm.at[wid], idx_v)

        def gather(j, b):
            return pltpu.make_async_copy(
                table_hbm.at[idx_v.at[j]], rows_v.at[b], gsem.at[b]
            )

        def outcp(j, b):
            return pltpu.make_async_copy(
                rows_v.at[b],
                out_hbm.at[pl.ds((wid * N_CHUNKS + j) * CHUNK, CHUNK)],
                osem.at[b],
            )

        for b in range(G):
            gather(b, b).start()

        def outer(g, carry):
            for b in range(NB):
                j = g * NB + b
                jn = j + G
                bn = (b + G) % NB

                @pl.when(jn < N_CHUNKS)
                def _():
                    @pl.when(jn >= NB)
                    def _():
                        outcp(jn - NB, bn).wait()

                    gather(jn, bn).start()

                gather(j, b).wait()

                def row_body(r, c2):
                    for c in range(DIM // 16):
                        sl = (b, r, pl.ds(c * 16, 16))
                        rows_v[sl] = rows_v[sl] * _SCALE
                    return c2

                lax.fori_loop(0, CHUNK, row_body, 0)
                outcp(j, b).start()
            return carry

        lax.fori_loop(0, N_CHUNKS // NB, outer, 0)

        for b in range(NB):
            outcp(N_CHUNKS - NB + b, b).wait()

    return lookup


_lookup = _make_lookup()


@jax.jit
def kernel(x, weight):
    idx = x.reshape(NW, N_CHUNKS, CHUNK)
    out = _lookup(idx, weight)
    return out.reshape(x.shape[0], x.shape[1], DIM)


# speedup vs baseline: 2.9670x; 1.0020x over previous
import functools
import math

import jax
import jax.numpy as jnp
from jax import lax
from jax.experimental import pallas as pl
from jax.experimental.pallas import tpu as pltpu
from jax.experimental.pallas import tpu_sc as plsc

NUM_EMB = 100000
DIM = 128
_SCALE = math.sqrt(DIM)

NC = 2
NS = 16
NW = NC * NS

B = 4096 * 50
B_PER_W = B // NW
CHUNK = 128
N_CHUNKS = B_PER_W // CHUNK
NB = 5
G = 4


def _make_lookup():
    mesh = plsc.VectorSubcoreMesh(core_axis_name="c", subcore_axis_name="s")

    @functools.partial(
        pl.kernel,
        mesh=mesh,
        out_type=jax.ShapeDtypeStruct((B, DIM), jnp.float32),
        scratch_types=[
            pltpu.VMEM((N_CHUNKS, CHUNK), jnp.int32),
            pltpu.VMEM((NB, CHUNK, DIM), jnp.float32),
            pltpu.SemaphoreType.DMA((NB,)),
            pltpu.SemaphoreType.DMA((NB,)),
        ],
    )
    def lookup(idx_hbm, table_hbm, out_hbm, idx_v, rows_v, gsem, osem):
        wid = lax.axis_index("s") * NC + lax.axis_index("c")
        pltpu.sync_copy(idx_hbm.at[wid], idx_v)

        def gather(j, b):
            return pltpu.make_async_copy(
                table_hbm.at[idx_v.at[j]], rows_v.at[b], gsem.at[b]
            )

        def outcp(j, b):
            return pltpu.make_async_copy(
                rows_v.at[b],
                out_hbm.at[pl.ds((wid * N_CHUNKS + j) * CHUNK, CHUNK)],
                osem.at[b],
            )

        for b in range(G):
            gather(b, b).start()

        def outer(g, carry):
            for b in range(NB):
                j = g * NB + b
                jn = j + G
                bn = (b + G) % NB

                @pl.when(jn < N_CHUNKS)
                def _():
                    @pl.when(jn >= NB)
                    def _():
                        outcp(jn - NB, bn).wait()

                    gather(jn, bn).start()

                gather(j, b).wait()
                rv = rows_v.at[b]

                def row_body(r, c2):
                    for c in range(DIM // 16):
                        sl = (r, pl.ds(c * 16, 16))
                        rv[sl] = rv[sl] * _SCALE
                    return c2

                lax.fori_loop(0, CHUNK, row_body, 0)
                outcp(j, b).start()
            return carry

        lax.fori_loop(0, N_CHUNKS // NB, outer, 0)

        for b in range(NB):
            outcp(N_CHUNKS - NB + b, b).wait()

    return lookup


_lookup = _make_lookup()


@jax.jit
def kernel(x, weight):
    idx = x.reshape(NW, N_CHUNKS, CHUNK)
    out = _lookup(idx, weight)
    return out.reshape(x.shape[0], x.shape[1], DIM)


# speedup vs baseline: 5.2608x; 1.7731x over previous
import functools
import math

import jax
import jax.numpy as jnp
from jax import lax
from jax.experimental import pallas as pl
from jax.experimental.pallas import tpu as pltpu
from jax.experimental.pallas import tpu_sc as plsc

NUM_EMB = 100000
DIM = 128
_SCALE = math.sqrt(DIM)

NC = 2
NS = 16
NW = NC * NS

BATCH = 4096
SEQ = 50
BR_PER_W = BATCH // NW
ROWS_PER_CHUNK = 2
CHUNK = ROWS_PER_CHUNK * SEQ
N_CHUNKS = BR_PER_W // ROWS_PER_CHUNK
NB = 4
G = 3


def _make_lookup():
    mesh = plsc.VectorSubcoreMesh(core_axis_name="c", subcore_axis_name="s")

    @functools.partial(
        pl.kernel,
        mesh=mesh,
        out_type=jax.ShapeDtypeStruct((BATCH, SEQ, DIM), jnp.float32),
        scratch_types=[
            pltpu.VMEM((N_CHUNKS, CHUNK), jnp.int32),
            pltpu.VMEM((NB, CHUNK, DIM), jnp.float32),
            pltpu.SemaphoreType.DMA((NB,)),
            pltpu.SemaphoreType.DMA((NB,)),
        ],
    )
    def lookup(idx_hbm, table_hbm, out_hbm, idx_v, rows_v, gsem, osem):
        wid = lax.axis_index("s") * NC + lax.axis_index("c")
        pltpu.sync_copy(idx_hbm.at[wid], idx_v)
        row_base = wid * BR_PER_W

        def gather(j, b):
            return pltpu.make_async_copy(
                table_hbm.at[idx_v.at[j]], rows_v.at[b], gsem.at[b]
            )

        def outcp(j, b, r):
            return pltpu.make_async_copy(
                rows_v.at[b].at[pl.ds(r * SEQ, SEQ)],
                out_hbm.at[row_base + j * ROWS_PER_CHUNK + r],
                osem.at[b],
            )

        for b in range(G):
            gather(b, b).start()

        def outer(g, carry):
            for b in range(NB):
                j = g * NB + b
                jn = j + G
                bn = (b + G) % NB

                @pl.when(jn < N_CHUNKS)
                def _():
                    @pl.when(jn >= NB)
                    def _():
                        for r in range(ROWS_PER_CHUNK):
                            outcp(jn - NB, bn, r).wait()

                    gather(jn, bn).start()

                gather(j, b).wait()
                rv = rows_v.at[b]

                def row_body(r, c2):
                    for c in range(DIM // 16):
                        sl = (r, pl.ds(c * 16, 16))
                        rv[sl] = rv[sl] * _SCALE
                    return c2

                lax.fori_loop(0, CHUNK, row_body, 0)
                for r in range(ROWS_PER_CHUNK):
                    outcp(j, b, r).start()
            return carry

        lax.fori_loop(0, N_CHUNKS // NB, outer, 0)

        for b in range(NB):
            for r in range(ROWS_PER_CHUNK):
                outcp(N_CHUNKS - NB + b, b, r).wait()

    return lookup


_lookup = _make_lookup()


@jax.jit
def kernel(x, weight):
    idx = x.reshape(NW, N_CHUNKS, CHUNK)
    return _lookup(idx, weight)


# speedup vs baseline: 9.0438x; 1.7191x over previous
import functools
import math

import jax
import jax.numpy as jnp
from jax import lax
from jax.experimental import pallas as pl
from jax.experimental.pallas import tpu as pltpu
from jax.experimental.pallas import tpu_sc as plsc

NUM_EMB = 100000
DIM = 128
_SCALE = math.sqrt(DIM)

NC = 2
NS = 16
NW = NC * NS

BATCH = 4096
SEQ = 50
CHUNK = BATCH // NW
N_CHUNKS = SEQ
NB = 5
G = 4


def _make_lookup():
    mesh = plsc.VectorSubcoreMesh(core_axis_name="c", subcore_axis_name="s")

    @functools.partial(
        pl.kernel,
        mesh=mesh,
        out_type=jax.ShapeDtypeStruct((SEQ, BATCH, DIM), jnp.float32),
        scratch_types=[
            pltpu.VMEM((N_CHUNKS, CHUNK), jnp.int32),
            pltpu.VMEM((NB, CHUNK, DIM), jnp.float32),
            pltpu.SemaphoreType.DMA((NB,)),
            pltpu.SemaphoreType.DMA((NB,)),
        ],
    )
    def lookup(idx_hbm, table_hbm, out_hbm, idx_v, rows_v, gsem, osem):
        wid = lax.axis_index("s") * NC + lax.axis_index("c")
        pltpu.sync_copy(idx_hbm.at[wid], idx_v)
        col0 = wid * CHUNK

        def gather(j, b):
            return pltpu.make_async_copy(
                table_hbm.at[idx_v.at[j]], rows_v.at[b], gsem.at[b]
            )

        def outcp(j, b):
            return pltpu.make_async_copy(
                rows_v.at[b],
                out_hbm.at[j].at[pl.ds(col0, CHUNK)],
                osem.at[b],
            )

        for b in range(G):
            gather(b, b).start()

        def outer(g, carry):
            for b in range(NB):
                j = g * NB + b
                jn = j + G
                bn = (b + G) % NB

                @pl.when(jn < N_CHUNKS)
                def _():
                    @pl.when(jn >= NB)
                    def _():
                        outcp(jn - NB, bn).wait()

                    gather(jn, bn).start()

                gather(j, b).wait()
                rv = rows_v.at[b]

                def row_body(r, c2):
                    for c in range(DIM // 16):
                        sl = (r, pl.ds(c * 16, 16))
                        rv[sl] = rv[sl] * _SCALE
                    return c2

                lax.fori_loop(0, CHUNK, row_body, 0)
                outcp(j, b).start()
            return carry

        lax.fori_loop(0, N_CHUNKS // NB, outer, 0)

        for b in range(NB):
            outcp(N_CHUNKS - NB + b, b).wait()

    return lookup


_lookup = _make_lookup()


@jax.jit
def kernel(x, weight):
    idx = x.T.reshape(SEQ, NW, CHUNK).transpose(1, 0, 2)
    out = _lookup(idx, weight)
    return out.transpose(1, 0, 2)
